# baseline (speedup 1.0000x reference)
import jax
import jax.numpy as jnp
from jax.experimental import pallas as pl

_R = 131072
_C = 128
_BR = 4096


def _copy_fix_body(x_ref, o_ref):
    x = x_ref[...]
    row = jax.lax.broadcasted_iota(jnp.int32, x.shape, 0)
    col = jax.lax.broadcasted_iota(jnp.int32, x.shape, 1)
    fix = jnp.logical_and(col == 0, (row % 2) == 0)
    o_ref[...] = jnp.where(fix, x[:, 1:2], x)


def kernel(b):
    b2 = b.reshape(_R, _C)
    out = pl.pallas_call(
        _copy_fix_body,
        grid=(_R // _BR,),
        in_specs=[pl.BlockSpec((_BR, _C), lambda i: (i, 0))],
        out_specs=pl.BlockSpec((_BR, _C), lambda i: (i, 0)),
        out_shape=jax.ShapeDtypeStruct((_R, _C), b.dtype),
    )(b2)
    return out.reshape(b.shape)


# speedup vs baseline: 1.1999x; 1.1999x over previous
import jax
import jax.numpy as jnp
from jax.experimental import pallas as pl

_R = 131072
_C = 128
_BR = 26624


def _copy_fix_body(x_ref, o_ref):
    x = x_ref[...]
    row = jax.lax.broadcasted_iota(jnp.int32, x.shape, 0)
    col = jax.lax.broadcasted_iota(jnp.int32, x.shape, 1)
    fix = jnp.logical_and(col == 0, (row % 2) == 0)
    o_ref[...] = jnp.where(fix, x[:, 1:2], x)


def kernel(b):
    b2 = b.reshape(_R, _C)
    out = pl.pallas_call(
        _copy_fix_body,
        grid=(pl.cdiv(_R, _BR),),
        in_specs=[pl.BlockSpec((_BR, _C), lambda i: (i, 0))],
        out_specs=pl.BlockSpec((_BR, _C), lambda i: (i, 0)),
        out_shape=jax.ShapeDtypeStruct((_R, _C), b.dtype),
    )(b2)
    return out.reshape(b.shape)


# speedup vs baseline: 1.2453x; 1.0378x over previous
import jax
import jax.numpy as jnp
from jax.experimental import pallas as pl

_R = 131072
_C = 128
_BR = 28672


def _copy_fix_body(x_ref, o_ref):
    x = x_ref[...]
    row = jax.lax.broadcasted_iota(jnp.int32, x.shape, 0)
    col = jax.lax.broadcasted_iota(jnp.int32, x.shape, 1)
    fix = jnp.logical_and(col == 0, (row % 2) == 0)
    o_ref[...] = jnp.where(fix, x[:, 1:2], x)


def kernel(b):
    b2 = b.reshape(_R, _C)
    out = pl.pallas_call(
        _copy_fix_body,
        grid=(pl.cdiv(_R, _BR),),
        in_specs=[pl.BlockSpec((_BR, _C), lambda i: (i, 0))],
        out_specs=pl.BlockSpec((_BR, _C), lambda i: (i, 0)),
        out_shape=jax.ShapeDtypeStruct((_R, _C), b.dtype),
    )(b2)
    return out.reshape(b.shape)
